# baseline (device time: 24293 ns/iter reference)
import jax
import jax.numpy as jnp
from jax import lax
from jax.experimental import pallas as pl
from jax.experimental.pallas import tpu as pltpu

N_DEV = 4
CHUNK = 16


def kernel(x):
    m, n = x.shape

    def body(x_ref, out_ref, tot_ref, send_sems, recv_sems):
        my_pos = lax.axis_index("i")

        barrier_sem = pltpu.get_barrier_semaphore()
        for k in range(1, N_DEV):
            peer = (my_pos + k) % N_DEV
            pl.semaphore_signal(
                barrier_sem, inc=1,
                device_id=(peer,), device_id_type=pl.DeviceIdType.MESH,
            )
        pl.semaphore_wait(barrier_sem, N_DEV - 1)

        xv = x_ref[...]
        g = m // CHUNK

        rowc = lax.broadcasted_iota(jnp.int32, (m, n), 0) % CHUNK
        acc = xv
        for d in (1, 2, 4, 8):
            rolled = pltpu.roll(acc, d, 0)
            acc = acc * jnp.where(rowc >= d, rolled, 1.0)

        c = acc.reshape(g, CHUNK, n)[:, CHUNK - 1, :]

        d = 1
        while d < g:
            shifted = jnp.concatenate(
                [jnp.ones((d, n), c.dtype), c[:-d, :]], axis=0
            )
            c = c * shifted
            d *= 2

        tot_ref[0, :, :] = c[g - 1:g, :]
        rdmas = []
        for k in range(1, N_DEV):
            rdma = pltpu.make_async_remote_copy(
                src_ref=tot_ref.at[0],
                dst_ref=tot_ref.at[k],
                send_sem=send_sems.at[k - 1],
                recv_sem=recv_sems.at[k - 1],
                device_id=((my_pos + k) % N_DEV,),
                device_id_type=pl.DeviceIdType.MESH,
            )
            rdma.start()
            rdmas.append(rdma)

        c_ex = jnp.concatenate([jnp.ones((1, n), c.dtype), c[:-1, :]], axis=0)
        ce_full = jnp.broadcast_to(c_ex[:, None, :], (g, CHUNK, n)).reshape(m, n)

        for rdma in rdmas:
            rdma.wait_send()
            rdma.wait_recv()

        prefix = jnp.ones((1, n), xv.dtype)
        for k in range(1, N_DEV):
            cond = ((my_pos - k) % N_DEV) < my_pos
            prefix = prefix * jnp.where(cond, tot_ref[k, :, :], 1.0)

        out_ref[...] = acc * ce_full * prefix

    return pl.pallas_call(
        body,
        out_shape=jax.ShapeDtypeStruct((m, n), x.dtype),
        in_specs=[pl.BlockSpec(memory_space=pltpu.VMEM)],
        out_specs=pl.BlockSpec(memory_space=pltpu.VMEM),
        scratch_shapes=[
            pltpu.VMEM((N_DEV, 1, n), x.dtype),
            pltpu.SemaphoreType.DMA((N_DEV - 1,)),
            pltpu.SemaphoreType.DMA((N_DEV - 1,)),
        ],
        compiler_params=pltpu.CompilerParams(collective_id=0),
    )(x)


# device time: 20802 ns/iter; 1.1678x vs baseline; 1.1678x over previous
import jax
import jax.numpy as jnp
from jax import lax
from jax.experimental import pallas as pl
from jax.experimental.pallas import tpu as pltpu

N_DEV = 4


def kernel(x):
    m, n = x.shape

    def body(x_ref, out_ref, tot_ref, send_sems, recv_sems):
        my_pos = lax.axis_index("i")

        barrier_sem = pltpu.get_barrier_semaphore()
        for k in range(1, N_DEV):
            peer = (my_pos + k) % N_DEV
            pl.semaphore_signal(
                barrier_sem, inc=1,
                device_id=(peer,), device_id_type=pl.DeviceIdType.MESH,
            )
        pl.semaphore_wait(barrier_sem, N_DEV - 1)

        acc = x_ref[...]
        d = 1
        while d < m // 2:
            shifted = jnp.concatenate(
                [jnp.ones((d, n), acc.dtype), acc[:-d, :]], axis=0
            )
            acc = acc * shifted
            d *= 2

        tot_ref[0, :, :] = acc[m // 2 - 1 : m // 2, :] * acc[m - 1 : m, :]
        rdmas = []
        for k in range(1, N_DEV):
            rdma = pltpu.make_async_remote_copy(
                src_ref=tot_ref.at[0],
                dst_ref=tot_ref.at[k],
                send_sem=send_sems.at[k - 1],
                recv_sem=recv_sems.at[k - 1],
                device_id=((my_pos + k) % N_DEV,),
                device_id_type=pl.DeviceIdType.MESH,
            )
            rdma.start()
            rdmas.append(rdma)

        shifted = jnp.concatenate(
            [jnp.ones((m // 2, n), acc.dtype), acc[: m // 2, :]], axis=0
        )

        for rdma in rdmas:
            rdma.wait_send()
            rdma.wait_recv()

        prefix = jnp.ones((1, n), acc.dtype)
        for k in range(1, N_DEV):
            cond = ((my_pos - k) % N_DEV) < my_pos
            prefix = prefix * jnp.where(cond, tot_ref[k, :, :], 1.0)

        out_ref[...] = acc * shifted * prefix

    return pl.pallas_call(
        body,
        out_shape=jax.ShapeDtypeStruct((m, n), x.dtype),
        in_specs=[pl.BlockSpec(memory_space=pltpu.VMEM)],
        out_specs=pl.BlockSpec(memory_space=pltpu.VMEM),
        scratch_shapes=[
            pltpu.VMEM((N_DEV, 1, n), x.dtype),
            pltpu.SemaphoreType.DMA((N_DEV - 1,)),
            pltpu.SemaphoreType.DMA((N_DEV - 1,)),
        ],
        compiler_params=pltpu.CompilerParams(collective_id=0),
    )(x)


# device time: 19874 ns/iter; 1.2224x vs baseline; 1.0467x over previous
import jax
import jax.numpy as jnp
from jax import lax
from jax.experimental import pallas as pl
from jax.experimental.pallas import tpu as pltpu

N_DEV = 4


def kernel(x):
    m, n = x.shape

    def body(x_ref, out_ref, tot_ref, send_sems, recv_sems):
        my_pos = lax.axis_index("i")

        barrier_sem = pltpu.get_barrier_semaphore()
        for k in range(1, N_DEV):
            peer = (my_pos + k) % N_DEV
            pl.semaphore_signal(
                barrier_sem, inc=1,
                device_id=(peer,), device_id_type=pl.DeviceIdType.MESH,
            )

        acc = x_ref[...]
        d = 1
        while d < m // 4:
            shifted = jnp.concatenate(
                [jnp.ones((d, n), acc.dtype), acc[:-d, :]], axis=0
            )
            acc = acc * shifted
            d *= 2

        q = m // 4
        tot_ref[0, :, :] = (
            acc[q - 1 : q, :] * acc[2 * q - 1 : 2 * q, :]
        ) * (acc[3 * q - 1 : 3 * q, :] * acc[4 * q - 1 : 4 * q, :])

        pl.semaphore_wait(barrier_sem, N_DEV - 1)
        rdmas = []
        for k in range(1, N_DEV):
            rdma = pltpu.make_async_remote_copy(
                src_ref=tot_ref.at[0],
                dst_ref=tot_ref.at[k],
                send_sem=send_sems.at[k - 1],
                recv_sem=recv_sems.at[k - 1],
                device_id=((my_pos + k) % N_DEV,),
                device_id_type=pl.DeviceIdType.MESH,
            )
            rdma.start()
            rdmas.append(rdma)

        shifted = jnp.concatenate(
            [jnp.ones((m // 4, n), acc.dtype), acc[: -(m // 4), :]], axis=0
        )
        acc = acc * shifted
        shifted = jnp.concatenate(
            [jnp.ones((m // 2, n), acc.dtype), acc[: m // 2, :]], axis=0
        )

        for rdma in rdmas:
            rdma.wait_send()
            rdma.wait_recv()

        prefix = jnp.ones((1, n), acc.dtype)
        for k in range(1, N_DEV):
            cond = ((my_pos - k) % N_DEV) < my_pos
            prefix = prefix * jnp.where(cond, tot_ref[k, :, :], 1.0)

        out_ref[...] = acc * shifted * prefix

    return pl.pallas_call(
        body,
        out_shape=jax.ShapeDtypeStruct((m, n), x.dtype),
        in_specs=[pl.BlockSpec(memory_space=pltpu.VMEM)],
        out_specs=pl.BlockSpec(memory_space=pltpu.VMEM),
        scratch_shapes=[
            pltpu.VMEM((N_DEV, 1, n), x.dtype),
            pltpu.SemaphoreType.DMA((N_DEV - 1,)),
            pltpu.SemaphoreType.DMA((N_DEV - 1,)),
        ],
        compiler_params=pltpu.CompilerParams(collective_id=0),
    )(x)
